# baseline (speedup 1.0000x reference)
import functools

import jax
import jax.numpy as jnp
from jax import lax
from jax.experimental import pallas as pl
from jax.experimental.pallas import tpu as pltpu
from jax.experimental.pallas import tpu_sc as plsc

_DIM = 64
_SLICE = 128
_CHUNK = 512


@functools.cache
def _build(total):
    info = plsc.get_sparse_core_info()
    num_cores, num_subcores = info.num_cores, info.num_subcores
    num_workers = num_cores * num_subcores
    per_worker = total // num_workers
    n_chunks = per_worker // _CHUNK
    mesh = plsc.VectorSubcoreMesh(core_axis_name="c", subcore_axis_name="s")

    @functools.partial(
        pl.kernel,
        mesh=mesh,
        compiler_params=pltpu.CompilerParams(use_tc_tiling_on_sc=False),
        out_type=jax.ShapeDtypeStruct((total, _DIM), jnp.float32),
        scratch_types=[
            pltpu.VMEM((per_worker,), jnp.int32),
            pltpu.VMEM((_CHUNK, _DIM), jnp.float32),
            pltpu.SemaphoreType.DMA,
        ],
    )
    def emb(table_hbm, idx_hbm, out_hbm, idx_v, rows_v, sem):
        wid = lax.axis_index("s") * num_cores + lax.axis_index("c")
        base = wid * per_worker
        pltpu.sync_copy(idx_hbm.at[pl.ds(base, per_worker)], idx_v)

        def body(g, carry):
            offs = g * _CHUNK
            copies = []
            for j in range(_CHUNK // _SLICE):
                copies.append(
                    pltpu.async_copy(
                        table_hbm.at[idx_v.at[pl.ds(offs + j * _SLICE, _SLICE)]],
                        rows_v.at[pl.ds(j * _SLICE, _SLICE)],
                        sem,
                    )
                )
            for c in copies:
                c.wait()
            pltpu.sync_copy(rows_v, out_hbm.at[pl.ds(base + offs, _CHUNK)])
            return carry

        lax.fori_loop(0, n_chunks, body, 0)

    return emb


def kernel(x, emb_weight):
    batch, hist = x.shape
    flat = x.reshape(batch * hist).astype(jnp.int32)
    out = _build(batch * hist)(emb_weight, flat)
    return out.reshape(batch, hist, _DIM)


# speedup vs baseline: 1.0284x; 1.0284x over previous
import functools

import jax
import jax.numpy as jnp
from jax import lax
from jax.experimental import pallas as pl
from jax.experimental.pallas import tpu as pltpu
from jax.experimental.pallas import tpu_sc as plsc

_DIM = 64
_SLICE = 128
_CHUNK = 512


@functools.cache
def _build(total):
    info = plsc.get_sparse_core_info()
    num_cores, num_subcores = info.num_cores, info.num_subcores
    num_workers = num_cores * num_subcores
    per_worker = total // num_workers
    n_chunks = per_worker // _CHUNK
    n_pairs = n_chunks // 2
    assert n_chunks % 2 == 0 and n_pairs >= 2
    mesh = plsc.VectorSubcoreMesh(core_axis_name="c", subcore_axis_name="s")

    @functools.partial(
        pl.kernel,
        mesh=mesh,
        compiler_params=pltpu.CompilerParams(use_tc_tiling_on_sc=False),
        out_type=jax.ShapeDtypeStruct((total, _DIM), jnp.float32),
        scratch_types=[
            pltpu.VMEM((per_worker,), jnp.int32),
            pltpu.VMEM((_CHUNK, _DIM), jnp.float32),
            pltpu.VMEM((_CHUNK, _DIM), jnp.float32),
            pltpu.SemaphoreType.DMA,
            pltpu.SemaphoreType.DMA,
            pltpu.SemaphoreType.DMA,
            pltpu.SemaphoreType.DMA,
        ],
    )
    def emb(table_hbm, idx_hbm, out_hbm, idx_v, rows0, rows1, sg0, sg1, so0, so1):
        wid = lax.axis_index("s") * num_cores + lax.axis_index("c")
        base = wid * per_worker
        pltpu.sync_copy(idx_hbm.at[pl.ds(base, per_worker)], idx_v)

        bufs = (rows0, rows1)
        sgs = (sg0, sg1)
        sos = (so0, so1)

        def gather(g, b, start):
            for j in range(_CHUNK // _SLICE):
                src = table_hbm.at[idx_v.at[pl.ds(g * _CHUNK + j * _SLICE, _SLICE)]]
                dst = bufs[b].at[pl.ds(j * _SLICE, _SLICE)]
                if start:
                    pltpu.async_copy(src, dst, sgs[b])
                else:
                    pltpu.make_async_copy(src, dst, sgs[b]).wait()

        def outcp(g, b, start):
            src = bufs[b]
            dst = out_hbm.at[pl.ds(base + g * _CHUNK, _CHUNK)]
            if start:
                pltpu.async_copy(src, dst, sos[b])
            else:
                pltpu.make_async_copy(src, dst, sos[b]).wait()

        gather(0, 0, True)
        gather(1, 1, True)
        gather(0, 0, False)
        outcp(0, 0, True)
        gather(1, 1, False)
        outcp(1, 1, True)
        outcp(0, 0, False)
        gather(2, 0, True)

        def body(k, carry):
            a = 2 * k
            outcp(a - 1, 1, False)
            gather(a + 1, 1, True)
            gather(a, 0, False)
            outcp(a, 0, True)
            gather(a + 1, 1, False)
            outcp(a + 1, 1, True)
            outcp(a, 0, False)
            gather(a + 2, 0, True)
            return carry

        lax.fori_loop(1, n_pairs - 1, body, 0)

        a = n_chunks - 2
        outcp(a - 1, 1, False)
        gather(a + 1, 1, True)
        gather(a, 0, False)
        outcp(a, 0, True)
        gather(a + 1, 1, False)
        outcp(a + 1, 1, True)
        outcp(a, 0, False)
        outcp(a + 1, 1, False)

    return emb


def kernel(x, emb_weight):
    batch, hist = x.shape
    flat = x.reshape(batch * hist).astype(jnp.int32)
    out = _build(batch * hist)(emb_weight, flat)
    return out.reshape(batch, hist, _DIM)
